# baseline (speedup 1.0000x reference)
import functools
import jax
import jax.numpy as jnp
from jax.experimental import pallas as pl
from jax.experimental.pallas import tpu as pltpu

B = 512
K = 512


def _scatter_body(obi_ref, isf_ref, rows_ref, dl_ref, ac_ref, out_ref):
    c = pl.program_id(0)
    a = ac_ref[0:1, :]
    cc = ac_ref[1:2, :]
    rows_n = rows_ref[...] * a + cc
    dl = dl_ref[0]
    oh = (jax.lax.broadcasted_iota(jnp.int32, (B, K), 0) == dl).astype(jnp.float32)
    part = jnp.dot(oh, rows_n, preferred_element_type=jnp.float32)

    @pl.when(isf_ref[c] == 1)
    def _():
        out_ref[...] = part

    @pl.when(isf_ref[c] == 0)
    def _():
        out_ref[...] += part


def _mlp_body(n_total, agg_ref, h_ref, cm_ref, w1_ref, w2_ref, out_ref, st_ref):
    i = pl.program_id(0)
    a = cm_ref[0:1, :]
    c = cm_ref[1:2, :]
    b1 = cm_ref[2:3, :]
    b2 = cm_ref[3:4, :]
    h0 = h_ref[...] * a + c
    t = agg_ref[...] + h0
    h1 = jnp.maximum(jnp.dot(t, w1_ref[...], preferred_element_type=jnp.float32) + b1, 0.0)
    h2 = jnp.maximum(jnp.dot(h1, w2_ref[...], preferred_element_type=jnp.float32) + b2, 0.0)
    ridx = i * B + jax.lax.broadcasted_iota(jnp.int32, (B, 1), 0)
    h2 = jnp.where(ridx < n_total, h2, 0.0)
    out_ref[...] = h2

    @pl.when(i == 0)
    def _():
        st_ref[...] = jnp.zeros_like(st_ref)

    st_ref[0:1, :] += jnp.sum(h2, axis=0, keepdims=True)
    st_ref[1:2, :] += jnp.sum(h2 * h2, axis=0, keepdims=True)


def _pool_body(n_graphs, h_ref, bid_ref, ac_ref, out_ref):
    i = pl.program_id(0)

    @pl.when(i == 0)
    def _():
        out_ref[...] = jnp.full_like(out_ref, -jnp.inf)

    a = ac_ref[0:1, :]
    c = ac_ref[1:2, :]
    rows = h_ref[...] * a + c
    bid = bid_ref[0]
    blo = jnp.min(bid)
    bhi = jnp.max(jnp.where(bid < n_graphs, bid, -1))

    def body(g, _):
        m = bid == g
        vals = jnp.where(m, rows, -jnp.inf)
        red = jnp.max(vals, axis=0, keepdims=True)
        out_ref[pl.ds(g, 1), :] = jnp.maximum(out_ref[pl.ds(g, 1), :], red)
        return 0

    jax.lax.fori_loop(blo, bhi + 1, body, 0)


def kernel(x, edge_index, batch,
           W1_0, b1_0, W2_0, b2_0, gamma_0, beta_0,
           W1_1, b1_1, W2_1, b2_1, gamma_1, beta_1,
           W1_2, b1_2, W2_2, b2_2, gamma_2, beta_2):
    N, D_IN = x.shape
    E = edge_index.shape[1]
    D = W2_0.shape[1]
    G = 2000
    NB = -(-N // B)
    NPAD = NB * B
    NC = -(-E // K) + NB
    EP = NC * K

    src = edge_index[0]
    dst = edge_index[1]

    order = jnp.argsort(dst)
    dst_s = dst[order]
    src_s = src[order]
    blk = dst_s // B
    cnt = jnp.bincount(blk, length=NB)
    cap = jnp.maximum(-(-cnt // K), 1) * K
    cum_cap = jnp.cumsum(cap)
    pad_off = cum_cap - cap
    blk_start = jnp.cumsum(cnt) - cnt
    p = pad_off[blk] + jnp.arange(E, dtype=jnp.int32) - blk_start[blk]
    src_pad = jnp.zeros((EP,), jnp.int32).at[p].set(src_s)
    dl_pad = jnp.full((EP,), -1, jnp.int32).at[p].set(dst_s - blk * B)
    dl3 = dl_pad.reshape(NC, 1, K)
    chunk_starts = jnp.arange(NC, dtype=jnp.int32) * K
    obi = jnp.minimum(jnp.searchsorted(cum_cap, chunk_starts, side='right'),
                      NB - 1).astype(jnp.int32)
    isf = jnp.concatenate([jnp.ones((1,), jnp.int32),
                           (obi[1:] != obi[:-1]).astype(jnp.int32)])
    bid3 = jnp.concatenate([batch, jnp.full((NPAD - N,), G, jnp.int32)]
                           ).reshape(NB, B, 1)

    x_pad = jnp.zeros((NPAD, D), x.dtype).at[:N, :D_IN].set(x)
    W1_0p = jnp.zeros((D, D), W1_0.dtype).at[:D_IN].set(W1_0)

    scatter_call = pl.pallas_call(
        _scatter_body,
        grid_spec=pltpu.PrefetchScalarGridSpec(
            num_scalar_prefetch=2,
            grid=(NC,),
            in_specs=[
                pl.BlockSpec((K, D), lambda c, obi_r, isf_r: (c, 0)),
                pl.BlockSpec((1, 1, K), lambda c, obi_r, isf_r: (c, 0, 0)),
                pl.BlockSpec((8, D), lambda c, obi_r, isf_r: (0, 0)),
            ],
            out_specs=pl.BlockSpec((B, D), lambda c, obi_r, isf_r: (obi_r[c], 0)),
        ),
        out_shape=jax.ShapeDtypeStruct((NPAD, D), jnp.float32),
    )

    mlp_call = pl.pallas_call(
        functools.partial(_mlp_body, N),
        grid=(NB,),
        in_specs=[
            pl.BlockSpec((B, D), lambda i: (i, 0)),
            pl.BlockSpec((B, D), lambda i: (i, 0)),
            pl.BlockSpec((8, D), lambda i: (0, 0)),
            pl.BlockSpec((D, D), lambda i: (0, 0)),
            pl.BlockSpec((D, D), lambda i: (0, 0)),
        ],
        out_specs=[
            pl.BlockSpec((B, D), lambda i: (i, 0)),
            pl.BlockSpec((8, D), lambda i: (0, 0)),
        ],
        out_shape=[
            jax.ShapeDtypeStruct((NPAD, D), jnp.float32),
            jax.ShapeDtypeStruct((8, D), jnp.float32),
        ],
    )

    pool_call = pl.pallas_call(
        functools.partial(_pool_body, G),
        grid=(NB,),
        in_specs=[
            pl.BlockSpec((B, D), lambda i: (i, 0)),
            pl.BlockSpec((1, B, 1), lambda i: (i, 0, 0)),
            pl.BlockSpec((8, D), lambda i: (0, 0)),
        ],
        out_specs=pl.BlockSpec((G, D), lambda i: (0, 0)),
        out_shape=jax.ShapeDtypeStruct((G, D), jnp.float32),
    )

    layers = [
        (W1_0p, b1_0, W2_0, b2_0, gamma_0, beta_0),
        (W1_1, b1_1, W2_1, b2_1, gamma_1, beta_1),
        (W1_2, b1_2, W2_2, b2_2, gamma_2, beta_2),
    ]

    h = x_pad
    a = jnp.ones((D,), jnp.float32)
    c = jnp.zeros((D,), jnp.float32)
    outs = []
    for (W1, b1, W2, b2, g, b) in layers:
        rows = jnp.take(h, src_pad, axis=0)
        ac8 = jnp.zeros((8, D), jnp.float32).at[0].set(a).at[1].set(c)
        agg = scatter_call(obi, isf, rows, dl3, ac8)
        cm = ac8.at[2].set(b1).at[3].set(b2)
        h2, st = mlp_call(agg, h, cm, W1, W2)
        mean = st[0] / N
        var = st[1] / N - mean * mean
        a = g / jnp.sqrt(var + 1e-5)
        c = b - mean * a
        ac_n = jnp.zeros((8, D), jnp.float32).at[0].set(a).at[1].set(c)
        outs.append(pool_call(h2, bid3, ac_n))
        h = h2
    return jnp.concatenate(outs, axis=1)
